# baseline (speedup 1.0000x reference)
import math

import jax
import jax.numpy as jnp
from jax.experimental import pallas as pl
from jax.experimental.pallas import tpu as pltpu

D_MODEL = 768
N_HEADS = 12
D_HEAD = D_MODEL // N_HEADS
L = 2048
U = max(1, min(L, int(0.6 * L)))
SCALE = 1.0 / math.sqrt(D_HEAD)
QCHUNK = 1024
NCHUNKS = L // QCHUNK
HI = jax.lax.Precision.DEFAULT


def _attn_body(x_ref, wq_ref, wk_ref, wv_ref, bq_ref, bk_ref, bv_ref,
               wo_ref, bo_ref, out_ref, s_buf, m_buf, sp_buf, spt_buf,
               o_all):
    h = pl.program_id(0)
    x = x_ref[...]
    k = jnp.dot(x, wk_ref[0], precision=HI,
                preferred_element_type=jnp.float32) + bk_ref[0]
    v = (jnp.dot(x, wv_ref[0], precision=HI,
                 preferred_element_type=jnp.float32)
         + bv_ref[0]).astype(jnp.bfloat16)
    ones_col = jnp.ones((L, 1), dtype=jnp.float32)
    ones_col_bf = jnp.ones((L, 1), dtype=jnp.bfloat16)

    def stats_body(c, _):
        base = c * QCHUNK
        xc = x_ref[pl.ds(base, QCHUNK), :]
        qc = (jnp.dot(xc, wq_ref[0], precision=HI,
                      preferred_element_type=jnp.float32) + bq_ref[0]) * SCALE
        sc = jax.lax.dot_general(qc, k, (((1,), (1,)), ((), ())),
                                 precision=HI,
                                 preferred_element_type=jnp.float32)
        s_buf[pl.ds(base, QCHUNK), :] = sc
        mc = jnp.max(sc, axis=1, keepdims=True)
        meanc = jnp.dot(sc, ones_col, precision=HI,
                        preferred_element_type=jnp.float32) * (1.0 / L)
        m_buf[pl.ds(base, QCHUNK), :] = mc
        spc = mc - meanc
        sp_buf[pl.ds(base, QCHUNK), :] = spc
        spt_buf[:, pl.ds(base, QCHUNK)] = spc.T
        return 0

    jax.lax.fori_loop(0, NCHUNKS, stats_body, 0)

    spt = spt_buf[...]

    def out_body(c, _):
        base = c * QCHUNK
        spc = sp_buf[pl.ds(base, QCHUNK), :]
        beats = (spt > spc).astype(jnp.float32)
        rank = jnp.dot(beats, ones_col, precision=HI,
                       preferred_element_type=jnp.float32)
        maskc = (rank < float(U)).astype(jnp.float32)
        sc = s_buf[pl.ds(base, QCHUNK), :]
        mc = m_buf[pl.ds(base, QCHUNK), :]
        pc = jnp.exp(sc - mc).astype(jnp.bfloat16)
        oc = jax.lax.dot_general(pc, v,
                                 (((1,), (0,)), ((), ())),
                                 precision=HI,
                                 preferred_element_type=jnp.float32)
        lc = jnp.dot(pc, ones_col_bf, precision=HI,
                     preferred_element_type=jnp.float32)
        o_all[h, pl.ds(base, QCHUNK), :] = oc * (maskc / lc)
        return 0

    jax.lax.fori_loop(0, NCHUNKS, out_body, 0)

    @pl.when(h == N_HEADS - 1)
    def _():
        o_full = jnp.concatenate(
            [o_all[i] for i in range(N_HEADS)], axis=1)
        out_ref[...] = jnp.dot(o_full, wo_ref[...], precision=HI,
                               preferred_element_type=jnp.float32) + bo_ref[...]


def kernel(x, Wq, bq, Wk, bk, Wv, bv, Wo, bo):
    x2 = x.reshape(L, D_MODEL)
    wqT = Wq.T.reshape(D_MODEL, N_HEADS, D_HEAD).transpose(1, 0, 2)
    wkT = Wk.T.reshape(D_MODEL, N_HEADS, D_HEAD).transpose(1, 0, 2)
    wvT = Wv.T.reshape(D_MODEL, N_HEADS, D_HEAD).transpose(1, 0, 2)
    woT = Wo.T
    bq3 = bq.reshape(N_HEADS, 1, D_HEAD)
    bk3 = bk.reshape(N_HEADS, 1, D_HEAD)
    bv3 = bv.reshape(N_HEADS, 1, D_HEAD)
    bo2 = bo.reshape(1, D_MODEL)

    out = pl.pallas_call(
        _attn_body,
        grid=(N_HEADS,),
        in_specs=[
            pl.BlockSpec((L, D_MODEL), lambda h: (0, 0)),
            pl.BlockSpec((1, D_MODEL, D_HEAD), lambda h: (h, 0, 0)),
            pl.BlockSpec((1, D_MODEL, D_HEAD), lambda h: (h, 0, 0)),
            pl.BlockSpec((1, D_MODEL, D_HEAD), lambda h: (h, 0, 0)),
            pl.BlockSpec((1, 1, D_HEAD), lambda h: (h, 0, 0)),
            pl.BlockSpec((1, 1, D_HEAD), lambda h: (h, 0, 0)),
            pl.BlockSpec((1, 1, D_HEAD), lambda h: (h, 0, 0)),
            pl.BlockSpec((D_MODEL, D_MODEL), lambda h: (0, 0)),
            pl.BlockSpec((1, D_MODEL), lambda h: (0, 0)),
        ],
        out_specs=pl.BlockSpec((L, D_MODEL), lambda h: (0, 0)),
        out_shape=jax.ShapeDtypeStruct((L, D_MODEL), jnp.float32),
        scratch_shapes=[
            pltpu.VMEM((L, L), jnp.float32),
            pltpu.VMEM((L, 1), jnp.float32),
            pltpu.VMEM((L, 1), jnp.float32),
            pltpu.VMEM((1, L), jnp.float32),
            pltpu.VMEM((N_HEADS, L, D_HEAD), jnp.float32),
        ],
    )(x2, wqT, wkT, wvT, bq3, bk3, bv3, woT, bo2)
    return out.reshape(1, L, D_MODEL)


# speedup vs baseline: 1.1297x; 1.1297x over previous
import math

import jax
import jax.numpy as jnp
from jax.experimental import pallas as pl
from jax.experimental.pallas import tpu as pltpu

D_MODEL = 768
N_HEADS = 12
D_HEAD = D_MODEL // N_HEADS
L = 2048
U = max(1, min(L, int(0.6 * L)))
SCALE = 1.0 / math.sqrt(D_HEAD)
QCHUNK = 1024
NCHUNKS = L // QCHUNK
HI = jax.lax.Precision.DEFAULT


def _attn_body(x_ref, wq_ref, wk_ref, wv_ref, bq_ref, bk_ref, bv_ref,
               wo_ref, bo_ref, out_ref, s_buf, m_buf, sp_buf, spt_buf,
               o_all):
    h = pl.program_id(0)
    x = x_ref[...]
    k = jnp.dot(x, wk_ref[0], precision=HI,
                preferred_element_type=jnp.float32) + bk_ref[0]
    v = (jnp.dot(x, wv_ref[0], precision=HI,
                 preferred_element_type=jnp.float32)
         + bv_ref[0]).astype(jnp.bfloat16)
    ones_col = jnp.ones((L, 1), dtype=jnp.float32)
    ones_col_bf = jnp.ones((L, 1), dtype=jnp.bfloat16)

    def stats_body(c, _):
        base = c * QCHUNK
        xc = x_ref[pl.ds(base, QCHUNK), :]
        qc = (jnp.dot(xc, wq_ref[0], precision=HI,
                      preferred_element_type=jnp.float32) + bq_ref[0]) * SCALE
        sc = jax.lax.dot_general(qc, k, (((1,), (1,)), ((), ())),
                                 precision=HI,
                                 preferred_element_type=jnp.float32)
        s_buf[pl.ds(base, QCHUNK), :] = sc
        mc = jnp.max(sc, axis=1, keepdims=True)
        meanc = jnp.sum(sc, axis=1, keepdims=True) * (1.0 / L)
        m_buf[pl.ds(base, QCHUNK), :] = mc
        spc = mc - meanc
        sp_buf[pl.ds(base, QCHUNK), :] = spc
        spt_buf[:, pl.ds(base, QCHUNK)] = spc.T
        return 0

    jax.lax.fori_loop(0, NCHUNKS, stats_body, 0)

    spt = spt_buf[...]

    def out_body(c, _):
        base = c * QCHUNK
        spc = sp_buf[pl.ds(base, QCHUNK), :]
        beats = (spt > spc).astype(jnp.float32)
        rank = jnp.dot(beats, ones_col, precision=HI,
                       preferred_element_type=jnp.float32)
        maskc = (rank < float(U)).astype(jnp.float32)
        sc = s_buf[pl.ds(base, QCHUNK), :]
        mc = m_buf[pl.ds(base, QCHUNK), :]
        pc = jnp.exp(sc - mc).astype(jnp.bfloat16)
        oc = jax.lax.dot_general(pc, v,
                                 (((1,), (0,)), ((), ())),
                                 precision=HI,
                                 preferred_element_type=jnp.float32)
        lc = jnp.dot(pc, ones_col_bf, precision=HI,
                     preferred_element_type=jnp.float32)
        o_all[h, pl.ds(base, QCHUNK), :] = oc * (maskc / lc)
        return 0

    jax.lax.fori_loop(0, NCHUNKS, out_body, 0)

    @pl.when(h == N_HEADS - 1)
    def _():
        o_full = jnp.concatenate(
            [o_all[i] for i in range(N_HEADS)], axis=1)
        out_ref[...] = jnp.dot(o_full, wo_ref[...], precision=HI,
                               preferred_element_type=jnp.float32) + bo_ref[...]


def kernel(x, Wq, bq, Wk, bk, Wv, bv, Wo, bo):
    x2 = x.reshape(L, D_MODEL)
    wqT = Wq.T.reshape(D_MODEL, N_HEADS, D_HEAD).transpose(1, 0, 2)
    wkT = Wk.T.reshape(D_MODEL, N_HEADS, D_HEAD).transpose(1, 0, 2)
    wvT = Wv.T.reshape(D_MODEL, N_HEADS, D_HEAD).transpose(1, 0, 2)
    woT = Wo.T
    bq3 = bq.reshape(N_HEADS, 1, D_HEAD)
    bk3 = bk.reshape(N_HEADS, 1, D_HEAD)
    bv3 = bv.reshape(N_HEADS, 1, D_HEAD)
    bo2 = bo.reshape(1, D_MODEL)

    out = pl.pallas_call(
        _attn_body,
        grid=(N_HEADS,),
        in_specs=[
            pl.BlockSpec((L, D_MODEL), lambda h: (0, 0)),
            pl.BlockSpec((1, D_MODEL, D_HEAD), lambda h: (h, 0, 0)),
            pl.BlockSpec((1, D_MODEL, D_HEAD), lambda h: (h, 0, 0)),
            pl.BlockSpec((1, D_MODEL, D_HEAD), lambda h: (h, 0, 0)),
            pl.BlockSpec((1, 1, D_HEAD), lambda h: (h, 0, 0)),
            pl.BlockSpec((1, 1, D_HEAD), lambda h: (h, 0, 0)),
            pl.BlockSpec((1, 1, D_HEAD), lambda h: (h, 0, 0)),
            pl.BlockSpec((D_MODEL, D_MODEL), lambda h: (0, 0)),
            pl.BlockSpec((1, D_MODEL), lambda h: (0, 0)),
        ],
        out_specs=pl.BlockSpec((L, D_MODEL), lambda h: (0, 0)),
        out_shape=jax.ShapeDtypeStruct((L, D_MODEL), jnp.float32),
        scratch_shapes=[
            pltpu.VMEM((L, L), jnp.float32),
            pltpu.VMEM((L, 1), jnp.float32),
            pltpu.VMEM((L, 1), jnp.float32),
            pltpu.VMEM((1, L), jnp.float32),
            pltpu.VMEM((N_HEADS, L, D_HEAD), jnp.float32),
        ],
    )(x2, wqT, wkT, wvT, bq3, bk3, bv3, woT, bo2)
    return out.reshape(1, L, D_MODEL)


# speedup vs baseline: 1.4946x; 1.3230x over previous
import math

import jax
import jax.numpy as jnp
from jax.experimental import pallas as pl
from jax.experimental.pallas import tpu as pltpu

D_MODEL = 768
N_HEADS = 12
D_HEAD = D_MODEL // N_HEADS
L = 2048
U = max(1, min(L, int(0.6 * L)))
SCALE = 1.0 / math.sqrt(D_HEAD)
QCHUNK = 1024
NCHUNKS = L // QCHUNK
HPS = 2
NSTEPS = N_HEADS // HPS
HI = jax.lax.Precision.DEFAULT


def _attn_body(x_ref, wq_ref, wk_ref, wv_ref, bq_ref, bk_ref, bv_ref,
               wo_ref, bo_ref, out_ref, s_buf, sp_buf, spt_buf,
               o_all, q_buf):
    g = pl.program_id(0)
    x = x_ref[...]
    k_all = jnp.dot(x, wk_ref[0], precision=HI,
                    preferred_element_type=jnp.float32) + bk_ref[0]
    v_all = (jnp.dot(x, wv_ref[0], precision=HI,
                     preferred_element_type=jnp.float32)
             + bv_ref[0]).astype(jnp.bfloat16)
    q_buf[...] = (jnp.dot(x, wq_ref[0], precision=HI,
                          preferred_element_type=jnp.float32)
                  + bq_ref[0]) * SCALE
    ones_col = jnp.ones((L, 1), dtype=jnp.float32)
    ones_bf = jnp.ones((L, 1), dtype=jnp.bfloat16)

    for s in range(HPS):
        k = k_all[:, s * D_HEAD:(s + 1) * D_HEAD]
        v_aug = jnp.concatenate(
            [v_all[:, s * D_HEAD:(s + 1) * D_HEAD], ones_bf], axis=1)

        def stats_body(c, _):
            base = c * QCHUNK
            qc = q_buf[pl.ds(base, QCHUNK), s * D_HEAD:(s + 1) * D_HEAD]
            sc = jax.lax.dot_general(qc, k, (((1,), (1,)), ((), ())),
                                     precision=HI,
                                     preferred_element_type=jnp.float32)
            mc = jnp.max(sc, axis=1, keepdims=True)
            meanc = jnp.sum(sc, axis=1, keepdims=True) * (1.0 / L)
            s_buf[pl.ds(base, QCHUNK), :] = (sc - mc).astype(jnp.bfloat16)
            spc = mc - meanc
            sp_buf[pl.ds(base, QCHUNK), :] = spc
            spt_buf[:, pl.ds(base, QCHUNK)] = spc.T
            return 0

        jax.lax.fori_loop(0, NCHUNKS, stats_body, 0)

        spt = spt_buf[...]

        def out_body(c, _):
            base = c * QCHUNK
            spc = sp_buf[pl.ds(base, QCHUNK), :]
            beats = (spt > spc).astype(jnp.float32)
            rank = jnp.dot(beats, ones_col, precision=HI,
                           preferred_element_type=jnp.float32)
            maskc = (rank < float(U)).astype(jnp.float32)
            pc = jnp.exp(s_buf[pl.ds(base, QCHUNK), :])
            oa = jax.lax.dot_general(pc, v_aug,
                                     (((1,), (0,)), ((), ())),
                                     precision=HI,
                                     preferred_element_type=jnp.float32)
            oc = oa[:, :D_HEAD]
            lc = oa[:, D_HEAD:D_HEAD + 1]
            o_all[g * HPS + s, pl.ds(base, QCHUNK), :] = oc * (maskc / lc)
            return 0

        jax.lax.fori_loop(0, NCHUNKS, out_body, 0)

    @pl.when(g == NSTEPS - 1)
    def _():
        o_full = jnp.concatenate(
            [o_all[i] for i in range(N_HEADS)], axis=1)
        out_ref[...] = jnp.dot(o_full, wo_ref[...], precision=HI,
                               preferred_element_type=jnp.float32) + bo_ref[...]


def kernel(x, Wq, bq, Wk, bk, Wv, bv, Wo, bo):
    x2 = x.reshape(L, D_MODEL)
    gw = D_HEAD * HPS
    wqT = Wq.T.reshape(D_MODEL, NSTEPS, gw).transpose(1, 0, 2)
    wkT = Wk.T.reshape(D_MODEL, NSTEPS, gw).transpose(1, 0, 2)
    wvT = Wv.T.reshape(D_MODEL, NSTEPS, gw).transpose(1, 0, 2)
    woT = Wo.T
    bq3 = bq.reshape(NSTEPS, 1, gw)
    bk3 = bk.reshape(NSTEPS, 1, gw)
    bv3 = bv.reshape(NSTEPS, 1, gw)
    bo2 = bo.reshape(1, D_MODEL)

    out = pl.pallas_call(
        _attn_body,
        grid=(NSTEPS,),
        in_specs=[
            pl.BlockSpec((L, D_MODEL), lambda g: (0, 0)),
            pl.BlockSpec((1, D_MODEL, gw), lambda g: (g, 0, 0)),
            pl.BlockSpec((1, D_MODEL, gw), lambda g: (g, 0, 0)),
            pl.BlockSpec((1, D_MODEL, gw), lambda g: (g, 0, 0)),
            pl.BlockSpec((1, 1, gw), lambda g: (g, 0, 0)),
            pl.BlockSpec((1, 1, gw), lambda g: (g, 0, 0)),
            pl.BlockSpec((1, 1, gw), lambda g: (g, 0, 0)),
            pl.BlockSpec((D_MODEL, D_MODEL), lambda g: (0, 0)),
            pl.BlockSpec((1, D_MODEL), lambda g: (0, 0)),
        ],
        out_specs=pl.BlockSpec((L, D_MODEL), lambda g: (0, 0)),
        out_shape=jax.ShapeDtypeStruct((L, D_MODEL), jnp.float32),
        scratch_shapes=[
            pltpu.VMEM((L, L), jnp.bfloat16),
            pltpu.VMEM((L, 1), jnp.float32),
            pltpu.VMEM((1, L), jnp.float32),
            pltpu.VMEM((N_HEADS, L, D_HEAD), jnp.float32),
            pltpu.VMEM((L, gw), jnp.float32),
        ],
    )(x2, wqT, wkT, wvT, bq3, bk3, bv3, woT, bo2)
    return out.reshape(1, L, D_MODEL)


# speedup vs baseline: 1.5692x; 1.0499x over previous
import math

import jax
import jax.numpy as jnp
from jax.experimental import pallas as pl
from jax.experimental.pallas import tpu as pltpu

D_MODEL = 768
N_HEADS = 12
D_HEAD = D_MODEL // N_HEADS
L = 2048
U = max(1, min(L, int(0.6 * L)))
SCALE = 1.0 / math.sqrt(D_HEAD)
QCHUNK = 1024
NCHUNKS = L // QCHUNK
HPS = 4
NSTEPS = N_HEADS // HPS
HI = jax.lax.Precision.DEFAULT


def _attn_body(x_ref, wq_ref, wk_ref, wv_ref, bq_ref, bk_ref, bv_ref,
               wo_ref, bo_ref, out_ref, s_buf, sp_buf, spt_buf,
               o_all, q_buf):
    g = pl.program_id(0)
    x = x_ref[...]
    k_all = jnp.dot(x, wk_ref[0], precision=HI,
                    preferred_element_type=jnp.float32) + bk_ref[0]
    v_all = (jnp.dot(x, wv_ref[0], precision=HI,
                     preferred_element_type=jnp.float32)
             + bv_ref[0]).astype(jnp.bfloat16)
    q_buf[...] = (jnp.dot(x, wq_ref[0], precision=HI,
                          preferred_element_type=jnp.float32)
                  + bq_ref[0]) * SCALE
    ones_col = jnp.ones((L, 1), dtype=jnp.float32)
    ones_bf = jnp.ones((L, 1), dtype=jnp.bfloat16)

    for s in range(HPS):
        k = k_all[:, s * D_HEAD:(s + 1) * D_HEAD]
        v_aug = jnp.concatenate(
            [v_all[:, s * D_HEAD:(s + 1) * D_HEAD], ones_bf], axis=1)

        def stats_body(c, _):
            base = c * QCHUNK
            qc = q_buf[pl.ds(base, QCHUNK), s * D_HEAD:(s + 1) * D_HEAD]
            sc = jax.lax.dot_general(qc, k, (((1,), (1,)), ((), ())),
                                     precision=HI,
                                     preferred_element_type=jnp.float32)
            mc = jnp.max(sc, axis=1, keepdims=True)
            meanc = jnp.sum(sc, axis=1, keepdims=True) * (1.0 / L)
            s_buf[pl.ds(base, QCHUNK), :] = (sc - mc).astype(jnp.bfloat16)
            spc = mc - meanc
            sp_buf[pl.ds(base, QCHUNK), :] = spc
            spt_buf[:, pl.ds(base, QCHUNK)] = spc.T
            return 0

        jax.lax.fori_loop(0, NCHUNKS, stats_body, 0)

        spt = spt_buf[...]

        def out_body(c, _):
            base = c * QCHUNK
            spc = sp_buf[pl.ds(base, QCHUNK), :]
            beats = (spt > spc).astype(jnp.float32)
            rank = jnp.dot(beats, ones_col, precision=HI,
                           preferred_element_type=jnp.float32)
            maskc = (rank < float(U)).astype(jnp.float32)
            pc = jnp.exp(s_buf[pl.ds(base, QCHUNK), :])
            oa = jax.lax.dot_general(pc, v_aug,
                                     (((1,), (0,)), ((), ())),
                                     precision=HI,
                                     preferred_element_type=jnp.float32)
            oc = oa[:, :D_HEAD]
            lc = oa[:, D_HEAD:D_HEAD + 1]
            o_all[g * HPS + s, pl.ds(base, QCHUNK), :] = oc * (maskc / lc)
            return 0

        jax.lax.fori_loop(0, NCHUNKS, out_body, 0)

    @pl.when(g == NSTEPS - 1)
    def _():
        o_full = jnp.concatenate(
            [o_all[i] for i in range(N_HEADS)], axis=1)
        out_ref[...] = jnp.dot(o_full, wo_ref[...], precision=HI,
                               preferred_element_type=jnp.float32) + bo_ref[...]


def kernel(x, Wq, bq, Wk, bk, Wv, bv, Wo, bo):
    x2 = x.reshape(L, D_MODEL)
    gw = D_HEAD * HPS
    wqT = Wq.T.reshape(D_MODEL, NSTEPS, gw).transpose(1, 0, 2)
    wkT = Wk.T.reshape(D_MODEL, NSTEPS, gw).transpose(1, 0, 2)
    wvT = Wv.T.reshape(D_MODEL, NSTEPS, gw).transpose(1, 0, 2)
    woT = Wo.T
    bq3 = bq.reshape(NSTEPS, 1, gw)
    bk3 = bk.reshape(NSTEPS, 1, gw)
    bv3 = bv.reshape(NSTEPS, 1, gw)
    bo2 = bo.reshape(1, D_MODEL)

    out = pl.pallas_call(
        _attn_body,
        grid=(NSTEPS,),
        in_specs=[
            pl.BlockSpec((L, D_MODEL), lambda g: (0, 0)),
            pl.BlockSpec((1, D_MODEL, gw), lambda g: (g, 0, 0)),
            pl.BlockSpec((1, D_MODEL, gw), lambda g: (g, 0, 0)),
            pl.BlockSpec((1, D_MODEL, gw), lambda g: (g, 0, 0)),
            pl.BlockSpec((1, 1, gw), lambda g: (g, 0, 0)),
            pl.BlockSpec((1, 1, gw), lambda g: (g, 0, 0)),
            pl.BlockSpec((1, 1, gw), lambda g: (g, 0, 0)),
            pl.BlockSpec((D_MODEL, D_MODEL), lambda g: (0, 0)),
            pl.BlockSpec((1, D_MODEL), lambda g: (0, 0)),
        ],
        out_specs=pl.BlockSpec((L, D_MODEL), lambda g: (0, 0)),
        out_shape=jax.ShapeDtypeStruct((L, D_MODEL), jnp.float32),
        scratch_shapes=[
            pltpu.VMEM((L, L), jnp.bfloat16),
            pltpu.VMEM((L, 1), jnp.float32),
            pltpu.VMEM((1, L), jnp.float32),
            pltpu.VMEM((N_HEADS, L, D_HEAD), jnp.float32),
            pltpu.VMEM((L, gw), jnp.float32),
        ],
    )(x2, wqT, wkT, wvT, bq3, bk3, bv3, woT, bo2)
    return out.reshape(1, L, D_MODEL)
